# baseline (speedup 1.0000x reference)
import functools

import jax
import jax.numpy as jnp
from jax.experimental import pallas as pl
from jax.experimental.pallas import tpu as pltpu

_TM = 128
_BIG = 1e30
_DIMS = (((0,), (0,)), ((), ()))


def _chamfer_body(p_ref, g_ref, out_ref, amin_ref, aval_ref, p2_ref, *,
                  n_col_tiles, inv_bn, inv_bm):
    b = pl.program_id(0)
    j = pl.program_id(1)

    p = p_ref[0]
    g = g_ref[0]

    @pl.when(j == 0)
    def _():
        q = p * p
        ones = jnp.ones((8, 128), jnp.float32)
        p2f = jax.lax.dot_general(
            q, ones, _DIMS, preferred_element_type=jnp.float32,
            precision=jax.lax.Precision.HIGHEST,
        )
        p2_ref[...] = p2f[:, 0:1]

    p2 = p2_ref[...]
    g2 = jnp.sum(g * g, axis=0, keepdims=True)

    cross_sel = jax.lax.dot_general(
        p, g, _DIMS, preferred_element_type=jnp.float32,
        precision=jax.lax.Precision.DEFAULT,
    )
    cross_ex = jax.lax.dot_general(
        p, g, _DIMS, preferred_element_type=jnp.float32,
        precision=jax.lax.Precision.HIGHEST,
    )
    d_sel = p2 - 2.0 * cross_sel + g2
    d_ex = p2 - 2.0 * cross_ex + g2

    @pl.when((b == 0) & (j == 0))
    def _():
        out_ref[...] = jnp.zeros_like(out_ref)

    cmin = jnp.min(d_sel, axis=0, keepdims=True)
    cval = jnp.min(jnp.where(d_sel == cmin, d_ex, _BIG), axis=0, keepdims=True)
    bsum = inv_bm * jnp.sum(jnp.sqrt(jnp.maximum(cval, 0.0) + 1e-8))
    out_ref[...] = out_ref[...] + bsum

    rmin = jnp.min(d_sel, axis=1, keepdims=True)
    rval = jnp.min(jnp.where(d_sel == rmin, d_ex, _BIG), axis=1, keepdims=True)

    @pl.when(j == 0)
    def _():
        amin_ref[...] = rmin
        aval_ref[...] = rval

    @pl.when(j > 0)
    def _():
        better = rmin < amin_ref[...]
        amin_ref[...] = jnp.where(better, rmin, amin_ref[...])
        aval_ref[...] = jnp.where(better, rval, aval_ref[...])

    @pl.when(j == n_col_tiles - 1)
    def _():
        fsum = inv_bn * jnp.sum(jnp.sqrt(jnp.maximum(aval_ref[...], 0.0) + 1e-8))
        out_ref[...] = out_ref[...] + fsum


@jax.jit
def kernel(predict_pc, gt_pc):
    bsz, _, n = predict_pc.shape
    m = gt_pc.shape[2]
    n_col_tiles = m // _TM

    p_pad = jnp.pad(predict_pc, ((0, 0), (0, 5), (0, 0)))
    g_pad = jnp.pad(gt_pc, ((0, 0), (0, 5), (0, 0)))

    body = functools.partial(
        _chamfer_body,
        n_col_tiles=n_col_tiles,
        inv_bn=1.0 / (bsz * n),
        inv_bm=1.0 / (bsz * m),
    )
    out = pl.pallas_call(
        body,
        grid=(bsz, n_col_tiles),
        in_specs=[
            pl.BlockSpec((1, 8, n), lambda b, j: (b, 0, 0)),
            pl.BlockSpec((1, 8, _TM), lambda b, j: (b, 0, j)),
        ],
        out_specs=pl.BlockSpec((1, 1), lambda b, j: (0, 0)),
        out_shape=jax.ShapeDtypeStruct((1, 1), jnp.float32),
        scratch_shapes=[
            pltpu.VMEM((n, 1), jnp.float32),
            pltpu.VMEM((n, 1), jnp.float32),
            pltpu.VMEM((n, 1), jnp.float32),
        ],
    )(p_pad, g_pad)
    return out[0, 0]


# speedup vs baseline: 1.5934x; 1.5934x over previous
import functools

import jax
import jax.numpy as jnp
from jax.experimental import pallas as pl
from jax.experimental.pallas import tpu as pltpu

_TM = 512
_RC = 2048
_BIG = 1e30
_DIMS = (((0,), (0,)), ((), ()))


def _chamfer_body(p_ref, g_ref, out_ref, amin_ref, aval_ref, p2_ref, *,
                  n, n_col_tiles, inv_bn, inv_bm):
    b = pl.program_id(0)
    j = pl.program_id(1)

    p = p_ref[0]
    g = g_ref[0]

    @pl.when(j == 0)
    def _():
        q = p * p
        ones = jnp.ones((8, 128), jnp.float32)
        p2f = jax.lax.dot_general(
            q, ones, _DIMS, preferred_element_type=jnp.float32,
            precision=jax.lax.Precision.HIGHEST,
        )
        p2_ref[...] = p2f[:, 0:1]

    g2 = jnp.sum(g * g, axis=0, keepdims=True)

    @pl.when((b == 0) & (j == 0))
    def _():
        out_ref[...] = jnp.zeros_like(out_ref)

    run_cmin = None
    run_cval = None
    for c in range(n // _RC):
        lo = c * _RC
        pc = p[:, lo:lo + _RC]
        cross_sel = jax.lax.dot_general(
            pc, g, _DIMS, preferred_element_type=jnp.float32,
            precision=jax.lax.Precision.DEFAULT,
        )
        cross_ex = jax.lax.dot_general(
            pc, g, _DIMS, preferred_element_type=jnp.float32,
            precision=jax.lax.Precision.HIGHEST,
        )
        p2c = p2_ref[lo:lo + _RC, :]
        d_sel = p2c - 2.0 * cross_sel + g2
        d_ex = p2c - 2.0 * cross_ex + g2

        rmin = jnp.min(d_sel, axis=1, keepdims=True)
        rval = jnp.min(jnp.where(d_sel == rmin, d_ex, _BIG),
                       axis=1, keepdims=True)

        @pl.when(j == 0)
        def _():
            amin_ref[lo:lo + _RC, :] = rmin
            aval_ref[lo:lo + _RC, :] = rval

        @pl.when(j > 0)
        def _():
            prev_min = amin_ref[lo:lo + _RC, :]
            prev_val = aval_ref[lo:lo + _RC, :]
            better = rmin < prev_min
            amin_ref[lo:lo + _RC, :] = jnp.where(better, rmin, prev_min)
            aval_ref[lo:lo + _RC, :] = jnp.where(better, rval, prev_val)

        cmin_c = jnp.min(d_sel, axis=0, keepdims=True)
        cval_c = jnp.min(jnp.where(d_sel == cmin_c, d_ex, _BIG),
                         axis=0, keepdims=True)
        if run_cmin is None:
            run_cmin, run_cval = cmin_c, cval_c
        else:
            better_c = cmin_c < run_cmin
            run_cval = jnp.where(better_c, cval_c, run_cval)
            run_cmin = jnp.minimum(cmin_c, run_cmin)

    bsum = inv_bm * jnp.sum(jnp.sqrt(jnp.maximum(run_cval, 0.0) + 1e-8))
    out_ref[...] = out_ref[...] + bsum

    @pl.when(j == n_col_tiles - 1)
    def _():
        fsum = inv_bn * jnp.sum(jnp.sqrt(jnp.maximum(aval_ref[...], 0.0) + 1e-8))
        out_ref[...] = out_ref[...] + fsum


@jax.jit
def kernel(predict_pc, gt_pc):
    bsz, _, n = predict_pc.shape
    m = gt_pc.shape[2]
    n_col_tiles = m // _TM

    p_pad = jnp.pad(predict_pc, ((0, 0), (0, 5), (0, 0)))
    g_pad = jnp.pad(gt_pc, ((0, 0), (0, 5), (0, 0)))

    body = functools.partial(
        _chamfer_body,
        n=n,
        n_col_tiles=n_col_tiles,
        inv_bn=1.0 / (bsz * n),
        inv_bm=1.0 / (bsz * m),
    )
    out = pl.pallas_call(
        body,
        grid=(bsz, n_col_tiles),
        in_specs=[
            pl.BlockSpec((1, 8, n), lambda b, j: (b, 0, 0)),
            pl.BlockSpec((1, 8, _TM), lambda b, j: (b, 0, j)),
        ],
        out_specs=pl.BlockSpec((1, 1), lambda b, j: (0, 0)),
        out_shape=jax.ShapeDtypeStruct((1, 1), jnp.float32),
        scratch_shapes=[
            pltpu.VMEM((n, 1), jnp.float32),
            pltpu.VMEM((n, 1), jnp.float32),
            pltpu.VMEM((n, 1), jnp.float32),
        ],
    )(p_pad, g_pad)
    return out[0, 0]


# speedup vs baseline: 2.4193x; 1.5183x over previous
import functools

import jax
import jax.numpy as jnp
from jax import lax
from jax.experimental import pallas as pl
from jax.experimental.pallas import tpu as pltpu
from jax.experimental.pallas import tpu_sc as plsc

_TM = 512
_RC = 2048
_IBIG = 2 ** 30
_DIMS = (((0,), (0,)), ((), ()))


def _select_body(p_ref, g_ref, fwd_ref, bwd_ref, amin_ref, aidx_ref, p2_ref,
                 *, n, m, bm_all, n_col_tiles):
    b = pl.program_id(0)
    j = pl.program_id(1)

    p = p_ref[0]
    g = g_ref[0]

    @pl.when(j == 0)
    def _():
        q = p * p
        ones = jnp.ones((8, 128), jnp.float32)
        p2f = jax.lax.dot_general(
            q, ones, _DIMS, preferred_element_type=jnp.float32,
            precision=jax.lax.Precision.HIGHEST,
        )
        p2_ref[...] = p2f[:, 0:1]

    g2 = jnp.sum(g * g, axis=0, keepdims=True)
    iota_col = lax.broadcasted_iota(jnp.int32, (_RC, _TM), 1)
    iota_row = lax.broadcasted_iota(jnp.int32, (_RC, _TM), 0)

    run_cmin = None
    run_cidx = None
    for c in range(n // _RC):
        lo = c * _RC
        pc = p[:, lo:lo + _RC]
        cross = jax.lax.dot_general(
            pc, g, _DIMS, preferred_element_type=jnp.float32,
            precision=jax.lax.Precision.DEFAULT,
        )
        p2c = p2_ref[lo:lo + _RC, :]
        d = p2c - 2.0 * cross + g2

        rmin = jnp.min(d, axis=1, keepdims=True)
        ridx = jnp.min(jnp.where(d == rmin, iota_col, _IBIG),
                       axis=1, keepdims=True) + j * _TM

        @pl.when(j == 0)
        def _():
            amin_ref[lo:lo + _RC, :] = rmin
            aidx_ref[lo:lo + _RC, :] = ridx

        @pl.when(j > 0)
        def _():
            prev_min = amin_ref[lo:lo + _RC, :]
            prev_idx = aidx_ref[lo:lo + _RC, :]
            better = rmin < prev_min
            amin_ref[lo:lo + _RC, :] = jnp.where(better, rmin, prev_min)
            aidx_ref[lo:lo + _RC, :] = jnp.where(better, ridx, prev_idx)

        cmin = jnp.min(d, axis=0, keepdims=True)
        cidx = jnp.min(jnp.where(d == cmin, iota_row, _IBIG),
                       axis=0, keepdims=True) + lo
        if run_cmin is None:
            run_cmin, run_cidx = cmin, cidx
        else:
            better_c = cmin < run_cmin
            run_cidx = jnp.where(better_c, cidx, run_cidx)
            run_cmin = jnp.minimum(cmin, run_cmin)

    bwd_ref[0, 0, :] = run_cidx[0, :] + (bm_all + b * n)

    @pl.when(j == n_col_tiles - 1)
    def _():
        fwd_ref[0] = aidx_ref[...] + b * m


def _loss_body(sel_ref, ref_ref, out_ref, *, fwd_steps, inv_bn, inv_bm):
    i = pl.program_id(0)
    s = sel_ref[...]
    r = ref_ref[...]
    diff = s - r
    row = jnp.sum(diff * diff, axis=1, keepdims=True)
    val = jnp.sqrt(row + 1e-8)
    scale = jnp.where(i < fwd_steps, inv_bn, inv_bm)

    @pl.when(i == 0)
    def _():
        out_ref[...] = jnp.zeros_like(out_ref)

    out_ref[...] = out_ref[...] + scale * jnp.sum(val)


def _sc_gather(table, idx_flat, rows, lanes):
    info = plsc.get_sparse_core_info()
    nw = info.num_cores * info.num_subcores
    per_w = rows // nw

    chunk = 256
    n_chunks = per_w // chunk
    mesh = plsc.VectorSubcoreMesh(core_axis_name="c", subcore_axis_name="s")

    @functools.partial(
        pl.kernel, mesh=mesh,
        out_type=jax.ShapeDtypeStruct((rows, lanes), jnp.float32),
        scratch_types=[
            pltpu.VMEM((per_w,), jnp.int32),
            pltpu.VMEM((chunk, lanes), jnp.float32),
            pltpu.SemaphoreType.DMA,
        ],
    )
    def gather_k(table_hbm, idx_hbm, out_hbm, idx_v, rows_v, sem):
        wid = lax.axis_index("s") * info.num_cores + lax.axis_index("c")
        base = wid * per_w
        pltpu.sync_copy(idx_hbm.at[pl.ds(base, per_w)], idx_v)
        for k in range(n_chunks):
            pltpu.async_copy(
                table_hbm.at[idx_v.at[pl.ds(k * chunk, chunk)]],
                rows_v, sem).wait()
            pltpu.sync_copy(rows_v, out_hbm.at[pl.ds(base + k * chunk, chunk)])

    return gather_k(table, idx_flat)


@jax.jit
def kernel(predict_pc, gt_pc):
    bsz, _, n = predict_pc.shape
    m = gt_pc.shape[2]
    n_col_tiles = m // _TM

    p_pad = jnp.pad(predict_pc, ((0, 0), (0, 5), (0, 0)))
    g_pad = jnp.pad(gt_pc, ((0, 0), (0, 5), (0, 0)))

    body = functools.partial(
        _select_body,
        n=n, m=m, bm_all=bsz * m, n_col_tiles=n_col_tiles,
    )
    fwd_idx, bwd_idx = pl.pallas_call(
        body,
        grid=(bsz, n_col_tiles),
        in_specs=[
            pl.BlockSpec((1, 8, n), lambda b, j: (b, 0, 0)),
            pl.BlockSpec((1, 8, _TM), lambda b, j: (b, 0, j)),
        ],
        out_specs=[
            pl.BlockSpec((1, n, 1), lambda b, j: (b, 0, 0)),
            pl.BlockSpec((1, 1, _TM), lambda b, j: (b, 0, j)),
        ],
        out_shape=[
            jax.ShapeDtypeStruct((bsz, n, 1), jnp.int32),
            jax.ShapeDtypeStruct((bsz, 1, m), jnp.int32),
        ],
        scratch_shapes=[
            pltpu.VMEM((n, 1), jnp.float32),
            pltpu.VMEM((n, 1), jnp.int32),
            pltpu.VMEM((n, 1), jnp.float32),
        ],
    )(p_pad, g_pad)

    lanes = 128
    g_rows = jnp.pad(
        jnp.transpose(gt_pc, (0, 2, 1)).reshape(bsz * m, 3),
        ((0, 0), (0, lanes - 3)))
    p_rows = jnp.pad(
        jnp.transpose(predict_pc, (0, 2, 1)).reshape(bsz * n, 3),
        ((0, 0), (0, lanes - 3)))
    table = jnp.concatenate([g_rows, p_rows], axis=0)
    ref_rows = jnp.concatenate([p_rows, g_rows], axis=0)
    idx_flat = jnp.concatenate(
        [fwd_idx.reshape(-1), bwd_idx.reshape(-1)])

    rows = idx_flat.shape[0]
    sel_rows = _sc_gather(table, idx_flat, rows, lanes)

    rl = 8192
    loss = functools.partial(
        _loss_body,
        fwd_steps=(bsz * n) // rl,
        inv_bn=1.0 / (bsz * n),
        inv_bm=1.0 / (bsz * m),
    )
    out = pl.pallas_call(
        loss,
        grid=(rows // rl,),
        in_specs=[
            pl.BlockSpec((rl, lanes), lambda i: (i, 0)),
            pl.BlockSpec((rl, lanes), lambda i: (i, 0)),
        ],
        out_specs=pl.BlockSpec((1, 1), lambda i: (0, 0)),
        out_shape=jax.ShapeDtypeStruct((1, 1), jnp.float32),
    )(sel_rows, ref_rows)
    return out[0, 0]


# speedup vs baseline: 2.9319x; 1.2119x over previous
import functools

import jax
import jax.numpy as jnp
from jax import lax
from jax.experimental import pallas as pl
from jax.experimental.pallas import tpu as pltpu
from jax.experimental.pallas import tpu_sc as plsc

_TM = 512
_RC = 2048
_FBIG = 1e9
_DIMS = (((0,), (0,)), ((), ()))


def _select_body(p_ref, g_ref, fwd_ref, bwd_ref, amin_ref, aidx_ref, p2_ref,
                 *, n, m, bm_all, n_col_tiles):
    b = pl.program_id(0)
    j = pl.program_id(1)

    p = p_ref[0]
    g = g_ref[0]

    @pl.when(j == 0)
    def _():
        q = p * p
        ones = jnp.full((8, 128), 0.25, jnp.float32)
        p2f = jax.lax.dot_general(
            q, ones, _DIMS, preferred_element_type=jnp.float32,
            precision=jax.lax.Precision.HIGHEST,
        )
        p2_ref[...] = p2f[:, 0:1]

    g2 = jnp.sum(g * g, axis=0, keepdims=True)
    iota_col = lax.broadcasted_iota(jnp.int32, (_RC, _TM), 1).astype(jnp.float32)
    iota_row = lax.broadcasted_iota(jnp.int32, (_RC, _TM), 0).astype(jnp.float32)

    run_cmin = None
    run_cidx = None
    for c in range(n // _RC):
        lo = c * _RC
        pc = p[:, lo:lo + _RC]
        mcross = jax.lax.dot_general(
            pc, g, _DIMS, preferred_element_type=jnp.float32,
            precision=jax.lax.Precision.DEFAULT,
        )
        p2c = p2_ref[lo:lo + _RC, :]
        d = (p2c + mcross) + g2

        rmin = jnp.min(d, axis=1, keepdims=True)
        ridx = jnp.min(jnp.where(d == rmin, iota_col, _FBIG),
                       axis=1, keepdims=True)

        @pl.when(j == 0)
        def _():
            amin_ref[lo:lo + _RC, :] = rmin
            aidx_ref[lo:lo + _RC, :] = ridx + (j * _TM).astype(jnp.float32)

        @pl.when(j > 0)
        def _():
            prev_min = amin_ref[lo:lo + _RC, :]
            prev_idx = aidx_ref[lo:lo + _RC, :]
            better = rmin < prev_min
            amin_ref[lo:lo + _RC, :] = jnp.where(better, rmin, prev_min)
            aidx_ref[lo:lo + _RC, :] = jnp.where(
                better, ridx + (j * _TM).astype(jnp.float32), prev_idx)

        cmin = jnp.min(d, axis=0, keepdims=True)
        cidx = jnp.min(jnp.where(d == cmin, iota_row, _FBIG),
                       axis=0, keepdims=True) + float(lo)
        if run_cmin is None:
            run_cmin, run_cidx = cmin, cidx
        else:
            better_c = cmin < run_cmin
            run_cidx = jnp.where(better_c, cidx, run_cidx)
            run_cmin = jnp.minimum(cmin, run_cmin)

    bwd_ref[0, 0, :] = run_cidx[0, :].astype(jnp.int32) + (bm_all + b * n)

    @pl.when(j == n_col_tiles - 1)
    def _():
        fwd_ref[0] = aidx_ref[...].astype(jnp.int32) + b * m


def _loss_body(sel_ref, ref_ref, out_ref, *, fwd_steps, inv_bn, inv_bm):
    i = pl.program_id(0)
    s = sel_ref[...]
    r = ref_ref[...]
    diff = s - r
    row = jnp.sum(diff * diff, axis=1, keepdims=True)
    val = jnp.sqrt(row + 1e-8)
    scale = jnp.where(i < fwd_steps, inv_bn, inv_bm)

    @pl.when(i == 0)
    def _():
        out_ref[...] = jnp.zeros_like(out_ref)

    out_ref[...] = out_ref[...] + scale * jnp.sum(val)


def _sc_gather(table, idx_flat, rows, lanes):
    info = plsc.get_sparse_core_info()
    nw = info.num_cores * info.num_subcores
    per_w = rows // nw

    chunk = 256
    n_chunks = per_w // chunk
    mesh = plsc.VectorSubcoreMesh(core_axis_name="c", subcore_axis_name="s")

    @functools.partial(
        pl.kernel, mesh=mesh,
        out_type=jax.ShapeDtypeStruct((rows, lanes), jnp.float32),
        scratch_types=[
            pltpu.VMEM((per_w,), jnp.int32),
            pltpu.VMEM((chunk, lanes), jnp.float32),
            pltpu.SemaphoreType.DMA,
        ],
    )
    def gather_k(table_hbm, idx_hbm, out_hbm, idx_v, rows_v, sem):
        wid = lax.axis_index("s") * info.num_cores + lax.axis_index("c")
        base = wid * per_w
        pltpu.sync_copy(idx_hbm.at[pl.ds(base, per_w)], idx_v)
        for k in range(n_chunks):
            pltpu.async_copy(
                table_hbm.at[idx_v.at[pl.ds(k * chunk, chunk)]],
                rows_v, sem).wait()
            pltpu.sync_copy(rows_v, out_hbm.at[pl.ds(base + k * chunk, chunk)])

    return gather_k(table, idx_flat)


@jax.jit
def kernel(predict_pc, gt_pc):
    bsz, _, n = predict_pc.shape
    m = gt_pc.shape[2]
    n_col_tiles = m // _TM

    p_pad = jnp.pad(-2.0 * predict_pc, ((0, 0), (0, 5), (0, 0)))
    g_pad = jnp.pad(gt_pc, ((0, 0), (0, 5), (0, 0)))

    body = functools.partial(
        _select_body,
        n=n, m=m, bm_all=bsz * m, n_col_tiles=n_col_tiles,
    )
    fwd_idx, bwd_idx = pl.pallas_call(
        body,
        grid=(bsz, n_col_tiles),
        in_specs=[
            pl.BlockSpec((1, 8, n), lambda b, j: (b, 0, 0)),
            pl.BlockSpec((1, 8, _TM), lambda b, j: (b, 0, j)),
        ],
        out_specs=[
            pl.BlockSpec((1, n, 1), lambda b, j: (b, 0, 0)),
            pl.BlockSpec((1, 1, _TM), lambda b, j: (b, 0, j)),
        ],
        out_shape=[
            jax.ShapeDtypeStruct((bsz, n, 1), jnp.int32),
            jax.ShapeDtypeStruct((bsz, 1, m), jnp.int32),
        ],
        scratch_shapes=[
            pltpu.VMEM((n, 1), jnp.float32),
            pltpu.VMEM((n, 1), jnp.float32),
            pltpu.VMEM((n, 1), jnp.float32),
        ],
    )(p_pad, g_pad)

    lanes = 128
    g_rows = jnp.pad(
        jnp.transpose(gt_pc, (0, 2, 1)).reshape(bsz * m, 3),
        ((0, 0), (0, lanes - 3)))
    p_rows = jnp.pad(
        jnp.transpose(predict_pc, (0, 2, 1)).reshape(bsz * n, 3),
        ((0, 0), (0, lanes - 3)))
    table = jnp.concatenate([g_rows, p_rows], axis=0)
    ref_rows = jnp.concatenate([p_rows, g_rows], axis=0)
    idx_flat = jnp.concatenate(
        [fwd_idx.reshape(-1), bwd_idx.reshape(-1)])

    rows = idx_flat.shape[0]
    sel_rows = _sc_gather(table, idx_flat, rows, lanes)

    rl = 8192
    loss = functools.partial(
        _loss_body,
        fwd_steps=(bsz * n) // rl,
        inv_bn=1.0 / (bsz * n),
        inv_bm=1.0 / (bsz * m),
    )
    out = pl.pallas_call(
        loss,
        grid=(rows // rl,),
        in_specs=[
            pl.BlockSpec((rl, lanes), lambda i: (i, 0)),
            pl.BlockSpec((rl, lanes), lambda i: (i, 0)),
        ],
        out_specs=pl.BlockSpec((1, 1), lambda i: (0, 0)),
        out_shape=jax.ShapeDtypeStruct((1, 1), jnp.float32),
    )(sel_rows, ref_rows)
    return out[0, 0]
